# Initial kernel scaffold; baseline (speedup 1.0000x reference)
import jax
import jax.numpy as jnp
from jax.experimental import pallas as pl


def kernel(memory, idx, values):
    raise NotImplementedError("write your pallas kernel here")


# speedup vs baseline: 2.9028x; 2.9028x over previous
import functools

import jax
import jax.numpy as jnp
from jax import lax
from jax.experimental import pallas as pl
from jax.experimental.pallas import tpu as pltpu
from jax.experimental.pallas import tpu_sc as plsc

_MOM = 0.9
_N = 1000000
_NPH = 4
_HN = _N // _NPH
_D = 64
_B = 16384
_NC = 2
_NS = 16
_L = 16

_TB = _B // _NS
_OB = _B // (_NC * _NS)
_PB = _OB // 4
_CH = 128
_TCH = _TB // _CH
_OCH = _OB // _CH
_PBITS = 14
_DUMP = 1024
_DMASK = _DUMP - 1


def _fori(n, fn):
    def body(j, carry):
        fn(j)
        return carry
    lax.fori_loop(0, n, body, jnp.int32(0))


def _body(mem_hbm, idx_hbm, val_hbm, out_hbm,
          tidx, tpos, sidx, twin, aux, cand, teff, widx, wwin, ridx, vidx,
          rows, vals, obuf, tbl, cnt, sem):
    c = lax.axis_index("c")
    s = lax.axis_index("s")
    wid = c * _NS + s
    tb = s * _TB
    ob = wid * _OB

    _fori(_TCH, lambda j: pltpu.sync_copy(
        idx_hbm.at[pl.ds(tb + j * _CH, _CH)], tidx.at[j]))
    _fori(_OCH, lambda j: pltpu.sync_copy(
        idx_hbm.at[pl.ds(ob + j * _CH, _CH)], widx.at[j]))

    def _ridx_body(j):
        for ci in range(_CH // _L):
            sl = pl.ds(ci * _L, _L)
            ridx[j, sl] = widx[j, sl] >> 1
    _fori(_OCH, _ridx_body)

    next_dma = pltpu.async_copy(mem_hbm.at[ridx.at[0]], rows, sem)

    iota = lax.iota(jnp.int32, _L)

    def _pos_body(j):
        for ci in range(_CH // _L):
            tpos[j, pl.ds(ci * _L, _L)] = (tb + j * _CH + ci * _L) + iota
    _fori(_TCH, _pos_body)

    for h in range(_NPH):
        lo = h * _HN
        hi = lo + _HN

        def _teff_body(j):
            for ci in range(_CH // _L):
                sl = pl.ds(ci * _L, _L)
                x = tidx[j, sl]
                p = tpos[j, sl]
                inr = (x >= lo) & (x < hi)
                cand[j, sl] = jnp.where(inr, 1, 0)
                teff[j, sl] = jnp.where(inr, x - lo, _HN + (p & _DMASK))
        _fori(_TCH, _teff_body)

        _fori(_TCH, lambda j: pltpu.sync_copy(tpos.at[j], tbl.at[teff.at[j]]))
        plsc.subcore_barrier()

        for _ in range(2):
            _fori(_TCH, lambda j: pltpu.sync_copy(tbl.at[teff.at[j]],
                                                  twin.at[j]))
            def _sidx_body(j):
                for ci in range(_CH // _L):
                    sl = pl.ds(ci * _L, _L)
                    p = tpos[j, sl]
                    t = twin[j, sl]
                    m = (cand[j, sl] == 1) & (p > t)
                    sidx[j, sl] = jnp.where(m, teff[j, sl], _HN + (p & _DMASK))
            _fori(_TCH, _sidx_body)
            _fori(_TCH, lambda j: pltpu.sync_copy(tpos.at[j],
                                                  tbl.at[sidx.at[j]]))
            plsc.subcore_barrier()

        base = plsc.fetch_and_add(cnt.at[0], 0, subcore_id=0)
        plsc.subcore_barrier()
        _fori(_TCH, lambda j: pltpu.sync_copy(tbl.at[teff.at[j]], twin.at[j]))
        def _cnt_body(j, acc):
            for ci in range(_CH // _L):
                sl = pl.ds(ci * _L, _L)
                beat = jnp.where(tpos[j, sl] > twin[j, sl], 1, 0)
                acc = acc + (beat & cand[j, sl])
            return acc
        cacc = lax.fori_loop(0, _TCH, _cnt_body,
                             jnp.zeros((_L,), jnp.int32))
        my_cnt = cacc[0]
        for k in range(1, _L):
            my_cnt = my_cnt + cacc[k]
        plsc.fetch_and_add(cnt.at[0], my_cnt, subcore_id=0)
        plsc.subcore_barrier()
        total = plsc.fetch_and_add(cnt.at[0], 0, subcore_id=0)
        plsc.subcore_barrier()

        @pl.when(total != base)
        def _fallback():
            def _am1_body(j):
                for ci in range(_CH // _L):
                    aux[j, pl.ds(ci * _L, _L)] = jnp.zeros((_L,), jnp.int32) - 1
            _fori(_TCH, _am1_body)
            _fori(_TCH, lambda j: pltpu.sync_copy(aux.at[j],
                                                  tbl.at[teff.at[j]]))
            plsc.subcore_barrier()
            def _az_body(j):
                for ci in range(_CH // _L):
                    aux[j, pl.ds(ci * _L, _L)] = jnp.zeros((_L,), jnp.int32)
            _fori(_TCH, _az_body)

            def _bit_round(it, m_carry):
                b = _PBITS - 1 - it
                def _fsc_body(j):
                    for ci in range(_CH // _L):
                        sl = pl.ds(ci * _L, _L)
                        p = tpos[j, sl]
                        mybit = (p >> b) & 1
                        wr = (cand[j, sl] & mybit) == 1
                        sidx[j, sl] = jnp.where(wr, teff[j, sl],
                                                _HN + (p & _DMASK))
                        twin[j, sl] = jnp.zeros((_L,), jnp.int32) + b
                _fori(_TCH, _fsc_body)
                _fori(_TCH, lambda j: pltpu.sync_copy(twin.at[j],
                                                      tbl.at[sidx.at[j]]))
                plsc.subcore_barrier()
                _fori(_TCH, lambda j: pltpu.sync_copy(tbl.at[teff.at[j]],
                                                      twin.at[j]))
                def _fup_body(j):
                    for ci in range(_CH // _L):
                        sl = pl.ds(ci * _L, _L)
                        p = tpos[j, sl]
                        mybit = (p >> b) & 1
                        hit = jnp.where(twin[j, sl] == b, 1, 0)
                        aux[j, sl] = aux[j, sl] | (hit << b)
                        cand[j, sl] = jnp.where(mybit == hit, cand[j, sl], 0)
                _fori(_TCH, _fup_body)
                plsc.subcore_barrier()
                return m_carry

            lax.fori_loop(0, _PBITS, _bit_round, jnp.int32(0))

            _fori(_TCH, lambda j: pltpu.sync_copy(aux.at[j],
                                                  tbl.at[teff.at[j]]))
            plsc.subcore_barrier()

        def _weff_body(j):
            for ci in range(_CH // _L):
                sl = pl.ds(ci * _L, _L)
                x = widx[j, sl]
                inr = (x >= lo) & (x < hi)
                vidx[j, sl] = jnp.where(inr, x - lo,
                                        _HN + (ci * _L + iota))
        _fori(_OCH, _weff_body)
        _fori(_OCH, lambda j: pltpu.sync_copy(tbl.at[vidx.at[j]], sidx.at[j]))
        def _wsel_body(j):
            for ci in range(_CH // _L):
                sl = pl.ds(ci * _L, _L)
                x = widx[j, sl]
                inr = (x >= lo) & (x < hi)
                wwin[j, sl] = jnp.where(inr, sidx[j, sl], wwin[j, sl])
        _fori(_OCH, _wsel_body)
        if h < _NPH - 1:
            plsc.subcore_barrier()

    def _vidx_body(j):
        for ci in range(_CH // _L):
            sl = pl.ds(ci * _L, _L)
            vidx[j, sl] = wwin[j, sl] >> 1
    _fori(_OCH, _vidx_body)

    for p in range(4):
        next_dma.wait()
        pltpu.sync_copy(val_hbm.at[vidx.at[p]], vals)

        def _blend_block(q, carry, _p=p):
            slq = pl.ds((q & 7) * _L, _L)
            parv = widx[_p, slq] & 1
            wparv = wwin[_p, slq] & 1
            for k in range(_L):
                pm = parv[k]
                wm = wparv[k]
                r = q * _L + k
                for ci in range(_D // _L):
                    a = rows[r, pl.ds(pm * _D + ci * _L, _L)]
                    b = vals[r, pl.ds(wm * _D + ci * _L, _L)]
                    obuf[r, pl.ds(ci * _L, _L)] = _MOM * a + (1.0 - _MOM) * b
            return carry

        lax.fori_loop(0, _PB // _L, _blend_block, jnp.int32(0))

        if p < 3:
            next_dma = pltpu.async_copy(mem_hbm.at[ridx.at[p + 1]], rows, sem)
        pltpu.sync_copy(obuf, out_hbm.at[pl.ds(ob + p * _PB, _PB)])


_sc_call = functools.partial(
    pl.kernel,
    out_type=jax.ShapeDtypeStruct((_B, _D), jnp.float32),
    mesh=plsc.VectorSubcoreMesh(core_axis_name="c", subcore_axis_name="s",
                                num_cores=_NC, num_subcores=_NS),
    scratch_types=[
        pltpu.VMEM((_TCH, _CH), jnp.int32),
        pltpu.VMEM((_TCH, _CH), jnp.int32),
        pltpu.VMEM((_TCH, _CH), jnp.int32),
        pltpu.VMEM((_TCH, _CH), jnp.int32),
        pltpu.VMEM((_TCH, _CH), jnp.int32),
        pltpu.VMEM((_TCH, _CH), jnp.int32),
        pltpu.VMEM((_TCH, _CH), jnp.int32),
        pltpu.VMEM((_OCH, _CH), jnp.int32),
        pltpu.VMEM((_OCH, _CH), jnp.int32),
        pltpu.VMEM((_OCH, _CH), jnp.int32),
        pltpu.VMEM((_OCH, _CH), jnp.int32),
        pltpu.VMEM((_CH, 2 * _D), jnp.float32),
        pltpu.VMEM((_CH, 2 * _D), jnp.float32),
        pltpu.VMEM((_CH, _D), jnp.float32),
        pltpu.VMEM_SHARED((_HN + _DUMP,), jnp.int32),
        pltpu.SMEM((1,), jnp.int32),
        pltpu.SemaphoreType.DMA,
    ],
)(_body)


def kernel(memory, idx, values):
    mem2 = memory.reshape(_N // 2, 2 * _D)
    val2 = values.reshape(_B // 2, 2 * _D)
    return _sc_call(mem2, idx.astype(jnp.int32), val2)


# speedup vs baseline: 2.9372x; 1.0119x over previous
import functools

import jax
import jax.numpy as jnp
from jax import lax
from jax.experimental import pallas as pl
from jax.experimental.pallas import tpu as pltpu
from jax.experimental.pallas import tpu_sc as plsc

_MOM = 0.9
_N = 1000000
_NPH = 4
_HN = _N // _NPH
_D = 64
_B = 16384
_NC = 2
_NS = 16
_L = 16

_TB = _B // _NS
_OB = _B // (_NC * _NS)
_PB = _OB // 4
_CH = 128
_TCH = _TB // _CH
_OCH = _OB // _CH
_PBITS = 14
_DUMP = 1024
_DMASK = _DUMP - 1


def _fori(n, fn):
    def body(j, carry):
        fn(j)
        return carry
    lax.fori_loop(0, n, body, jnp.int32(0))


def _body(mem_hbm, idx_hbm, val_hbm, out_hbm,
          tidx, tpos, sidx, twin, aux, cand, teff, widx, wwin, vidx,
          rows, vals, tbl, cnt, sem):
    c = lax.axis_index("c")
    s = lax.axis_index("s")
    wid = c * _NS + s
    tb = s * _TB
    ob = wid * _OB

    _fori(_TCH, lambda j: pltpu.sync_copy(
        idx_hbm.at[pl.ds(tb + j * _CH, _CH)], tidx.at[j]))
    _fori(_OCH, lambda j: pltpu.sync_copy(
        idx_hbm.at[pl.ds(ob + j * _CH, _CH)], widx.at[j]))

    next_dma = pltpu.async_copy(mem_hbm.at[widx.at[0]], rows, sem)

    iota = lax.iota(jnp.int32, _L)

    def _pos_body(j):
        for ci in range(_CH // _L):
            tpos[j, pl.ds(ci * _L, _L)] = (tb + j * _CH + ci * _L) + iota
    _fori(_TCH, _pos_body)

    for h in range(_NPH):
        lo = h * _HN
        hi = lo + _HN

        def _teff_body(j):
            for ci in range(_CH // _L):
                sl = pl.ds(ci * _L, _L)
                x = tidx[j, sl]
                p = tpos[j, sl]
                inr = (x >= lo) & (x < hi)
                cand[j, sl] = jnp.where(inr, 1, 0)
                teff[j, sl] = jnp.where(inr, x - lo, _HN + (p & _DMASK))
        _fori(_TCH, _teff_body)

        _fori(_TCH, lambda j: pltpu.sync_copy(tpos.at[j], tbl.at[teff.at[j]]))
        plsc.subcore_barrier()

        for _ in range(2):
            _fori(_TCH, lambda j: pltpu.sync_copy(tbl.at[teff.at[j]],
                                                  twin.at[j]))
            def _sidx_body(j):
                for ci in range(_CH // _L):
                    sl = pl.ds(ci * _L, _L)
                    p = tpos[j, sl]
                    t = twin[j, sl]
                    m = (cand[j, sl] == 1) & (p > t)
                    sidx[j, sl] = jnp.where(m, teff[j, sl], _HN + (p & _DMASK))
            _fori(_TCH, _sidx_body)
            _fori(_TCH, lambda j: pltpu.sync_copy(tpos.at[j],
                                                  tbl.at[sidx.at[j]]))
            plsc.subcore_barrier()

        base = plsc.fetch_and_add(cnt.at[0], 0, subcore_id=0)
        plsc.subcore_barrier()
        _fori(_TCH, lambda j: pltpu.sync_copy(tbl.at[teff.at[j]], twin.at[j]))
        def _cnt_body(j, acc):
            for ci in range(_CH // _L):
                sl = pl.ds(ci * _L, _L)
                beat = jnp.where(tpos[j, sl] > twin[j, sl], 1, 0)
                acc = acc + (beat & cand[j, sl])
            return acc
        cacc = lax.fori_loop(0, _TCH, _cnt_body,
                             jnp.zeros((_L,), jnp.int32))
        my_cnt = cacc[0]
        for k in range(1, _L):
            my_cnt = my_cnt + cacc[k]
        plsc.fetch_and_add(cnt.at[0], my_cnt, subcore_id=0)
        plsc.subcore_barrier()
        total = plsc.fetch_and_add(cnt.at[0], 0, subcore_id=0)
        plsc.subcore_barrier()

        @pl.when(total != base)
        def _fallback():
            def _am1_body(j):
                for ci in range(_CH // _L):
                    aux[j, pl.ds(ci * _L, _L)] = jnp.zeros((_L,), jnp.int32) - 1
            _fori(_TCH, _am1_body)
            _fori(_TCH, lambda j: pltpu.sync_copy(aux.at[j],
                                                  tbl.at[teff.at[j]]))
            plsc.subcore_barrier()
            def _az_body(j):
                for ci in range(_CH // _L):
                    aux[j, pl.ds(ci * _L, _L)] = jnp.zeros((_L,), jnp.int32)
            _fori(_TCH, _az_body)

            def _bit_round(it, m_carry):
                b = _PBITS - 1 - it
                def _fsc_body(j):
                    for ci in range(_CH // _L):
                        sl = pl.ds(ci * _L, _L)
                        p = tpos[j, sl]
                        mybit = (p >> b) & 1
                        wr = (cand[j, sl] & mybit) == 1
                        sidx[j, sl] = jnp.where(wr, teff[j, sl],
                                                _HN + (p & _DMASK))
                        twin[j, sl] = jnp.zeros((_L,), jnp.int32) + b
                _fori(_TCH, _fsc_body)
                _fori(_TCH, lambda j: pltpu.sync_copy(twin.at[j],
                                                      tbl.at[sidx.at[j]]))
                plsc.subcore_barrier()
                _fori(_TCH, lambda j: pltpu.sync_copy(tbl.at[teff.at[j]],
                                                      twin.at[j]))
                def _fup_body(j):
                    for ci in range(_CH // _L):
                        sl = pl.ds(ci * _L, _L)
                        p = tpos[j, sl]
                        mybit = (p >> b) & 1
                        hit = jnp.where(twin[j, sl] == b, 1, 0)
                        aux[j, sl] = aux[j, sl] | (hit << b)
                        cand[j, sl] = jnp.where(mybit == hit, cand[j, sl], 0)
                _fori(_TCH, _fup_body)
                plsc.subcore_barrier()
                return m_carry

            lax.fori_loop(0, _PBITS, _bit_round, jnp.int32(0))

            _fori(_TCH, lambda j: pltpu.sync_copy(aux.at[j],
                                                  tbl.at[teff.at[j]]))
            plsc.subcore_barrier()

        def _weff_body(j):
            for ci in range(_CH // _L):
                sl = pl.ds(ci * _L, _L)
                x = widx[j, sl]
                inr = (x >= lo) & (x < hi)
                vidx[j, sl] = jnp.where(inr, x - lo,
                                        _HN + (ci * _L + iota))
        _fori(_OCH, _weff_body)
        _fori(_OCH, lambda j: pltpu.sync_copy(tbl.at[vidx.at[j]], sidx.at[j]))
        def _wsel_body(j):
            for ci in range(_CH // _L):
                sl = pl.ds(ci * _L, _L)
                x = widx[j, sl]
                inr = (x >= lo) & (x < hi)
                wwin[j, sl] = jnp.where(inr, sidx[j, sl], wwin[j, sl])
        _fori(_OCH, _wsel_body)
        if h < _NPH - 1:
            plsc.subcore_barrier()

    for p in range(4):
        next_dma.wait()
        pltpu.sync_copy(val_hbm.at[wwin.at[p]], vals)

        def _blend_block(q, carry):
            for k in range(_L):
                r = q * _L + k
                for ci in range(_D // _L):
                    sl = pl.ds(ci * _L, _L)
                    rows[r, sl] = _MOM * rows[r, sl] + (1.0 - _MOM) * vals[r, sl]
            return carry

        lax.fori_loop(0, _PB // _L, _blend_block, jnp.int32(0))

        pltpu.sync_copy(rows, out_hbm.at[pl.ds(ob + p * _PB, _PB)])
        if p < 3:
            next_dma = pltpu.async_copy(mem_hbm.at[widx.at[p + 1]], rows, sem)


_sc_call = functools.partial(
    pl.kernel,
    out_type=jax.ShapeDtypeStruct((_B, _D), jnp.float32),
    compiler_params=pltpu.CompilerParams(use_tc_tiling_on_sc=False),
    mesh=plsc.VectorSubcoreMesh(core_axis_name="c", subcore_axis_name="s",
                                num_cores=_NC, num_subcores=_NS),
    scratch_types=[
        pltpu.VMEM((_TCH, _CH), jnp.int32),
        pltpu.VMEM((_TCH, _CH), jnp.int32),
        pltpu.VMEM((_TCH, _CH), jnp.int32),
        pltpu.VMEM((_TCH, _CH), jnp.int32),
        pltpu.VMEM((_TCH, _CH), jnp.int32),
        pltpu.VMEM((_TCH, _CH), jnp.int32),
        pltpu.VMEM((_TCH, _CH), jnp.int32),
        pltpu.VMEM((_OCH, _CH), jnp.int32),
        pltpu.VMEM((_OCH, _CH), jnp.int32),
        pltpu.VMEM((_OCH, _CH), jnp.int32),
        pltpu.VMEM((_CH, _D), jnp.float32),
        pltpu.VMEM((_CH, _D), jnp.float32),
        pltpu.VMEM_SHARED((_HN + _DUMP,), jnp.int32),
        pltpu.SMEM((1,), jnp.int32),
        pltpu.SemaphoreType.DMA,
    ],
)(_body)


def kernel(memory, idx, values):
    return _sc_call(memory, idx.astype(jnp.int32), values)
